# Initial kernel scaffold; baseline (speedup 1.0000x reference)
import jax
import jax.numpy as jnp
from jax.experimental import pallas as pl


def kernel(nodes, edges, edge_index, edge_pair_index, edge_pair_node, nodeInt_params, edgeInt_params, nodeUpdate_params, edgeUpdate_params):
    raise NotImplementedError("write your pallas kernel here")


# speedup vs baseline: 2.5808x; 2.5808x over previous
import functools

import jax
import jax.numpy as jnp
from jax import lax
from jax.experimental import pallas as pl
from jax.experimental.pallas import tpu as pltpu
from jax.experimental.pallas import tpu_sc as plsc

NC = 2
NS = 16
NW = NC * NS
CK = 128

f32 = jnp.float32
i32 = jnp.int32


def _sc_gather(nodes, edges_pad, edge_len, src, dst, epn, e0, e1):
    n_nodes, node_len = nodes.shape
    n_edges = edges_pad.shape[0]
    nchunk = n_edges // CK
    iters = pl.cdiv(nchunk, NW)
    mesh = plsc.VectorSubcoreMesh(core_axis_name="c", subcore_axis_name="s")

    def body(nodes_h, edges_h, src_h, dst_h, epn_h, e0_h, e1_h,
             srcN_h, dstN_h, epnN_h, e0E_h, e1E_h,
             isrc, idst, iepn, ie0, ie1,
             rsrc, rdst, repn, re0, re1, ce0, ce1, sem):
        wid = lax.axis_index("s") * NC + lax.axis_index("c")

        def step(j, carry):
            c = j * NW + wid

            @pl.when(c < nchunk)
            def _():
                base = c * CK
                cps = [pltpu.async_copy(src_h.at[pl.ds(base, CK)], isrc, sem),
                       pltpu.async_copy(dst_h.at[pl.ds(base, CK)], idst, sem),
                       pltpu.async_copy(epn_h.at[pl.ds(base, CK)], iepn, sem),
                       pltpu.async_copy(e0_h.at[pl.ds(base, CK)], ie0, sem),
                       pltpu.async_copy(e1_h.at[pl.ds(base, CK)], ie1, sem)]
                for cp in cps:
                    cp.wait()
                cps = [pltpu.async_copy(nodes_h.at[isrc], rsrc, sem),
                       pltpu.async_copy(nodes_h.at[idst], rdst, sem),
                       pltpu.async_copy(nodes_h.at[iepn], repn, sem),
                       pltpu.async_copy(edges_h.at[ie0], re0, sem),
                       pltpu.async_copy(edges_h.at[ie1], re1, sem)]
                for cp in cps:
                    cp.wait()

                def compact(r, carry2):
                    ce0[r, :] = re0[r, pl.ds(0, edge_len)]
                    ce1[r, :] = re1[r, pl.ds(0, edge_len)]
                    return carry2

                lax.fori_loop(0, CK, compact, None)
                cps = [pltpu.async_copy(rsrc, srcN_h.at[pl.ds(base, CK)], sem),
                       pltpu.async_copy(rdst, dstN_h.at[pl.ds(base, CK)], sem),
                       pltpu.async_copy(repn, epnN_h.at[pl.ds(base, CK)], sem),
                       pltpu.async_copy(ce0, e0E_h.at[pl.ds(base, CK)], sem),
                       pltpu.async_copy(ce1, e1E_h.at[pl.ds(base, CK)], sem)]
                for cp in cps:
                    cp.wait()

            return carry

        lax.fori_loop(0, iters, step, None)

    out_type = (jax.ShapeDtypeStruct((n_edges, node_len), f32),
                jax.ShapeDtypeStruct((n_edges, node_len), f32),
                jax.ShapeDtypeStruct((n_edges, node_len), f32),
                jax.ShapeDtypeStruct((n_edges, edge_len), f32),
                jax.ShapeDtypeStruct((n_edges, edge_len), f32))
    scratch = [pltpu.VMEM((CK,), i32)] * 5 + \
              [pltpu.VMEM((CK, node_len), f32)] * 5 + \
              [pltpu.VMEM((CK, edge_len), f32)] * 2 + \
              [pltpu.SemaphoreType.DMA]
    return pl.kernel(body, out_type=out_type, mesh=mesh,
                     scratch_types=scratch)(nodes, edges_pad, src, dst,
                                            epn, e0, e1)


def _sc_scatter_node(vec, dst, n_nodes):
    n_edges, d = vec.shape
    nchunk = n_edges // CK
    iters = pl.cdiv(nchunk, NW)
    zone = ((n_nodes + NS * 8 - 1) // (NS * 8)) * 8
    n_pad = zone * NS
    zeros = jnp.zeros((zone, d), f32)
    mesh = plsc.VectorSubcoreMesh(core_axis_name="c", subcore_axis_name="s")

    def body(vec_h, dst_h, z_h, out_h, idx_v, vec_v, buf, sem):
        cid = lax.axis_index("c")
        sid = lax.axis_index("s")
        wid = sid * NC + cid
        pltpu.sync_copy(z_h, buf.at[pl.ds(sid * zone, zone)])
        plsc.subcore_barrier()

        def step(j, carry):
            c = j * NW + wid

            @pl.when(c < nchunk)
            def _():
                base = c * CK
                cp1 = pltpu.async_copy(dst_h.at[pl.ds(base, CK)], idx_v, sem)
                cp2 = pltpu.async_copy(vec_h.at[pl.ds(base, CK)], vec_v, sem)
                cp1.wait()
                cp2.wait()
                pltpu.sync_copy(vec_v, buf.at[idx_v], add=True)

            return carry

        lax.fori_loop(0, iters, step, None)
        plsc.subcore_barrier()
        pltpu.sync_copy(buf.at[pl.ds(sid * zone, zone)],
                        out_h.at[cid, pl.ds(sid * zone, zone)])

    out_type = jax.ShapeDtypeStruct((NC, n_pad, d), f32)
    scratch = [pltpu.VMEM((CK,), i32),
               pltpu.VMEM((CK, d), f32),
               pltpu.VMEM_SHARED((n_pad, d), f32),
               pltpu.SemaphoreType.DMA]
    out = pl.kernel(body, out_type=out_type, mesh=mesh,
                    scratch_types=scratch)(vec, dst, zeros)
    return out[:, :n_nodes, :]


def _sc_scatter_edge(vec_placed, e0, d, n_ranges=4):
    n_edges, dp = vec_placed.shape
    nchunk = n_edges // CK
    iters = pl.cdiv(nchunk, NS)
    p_rows = n_edges // 4
    rng_rows = p_rows // n_ranges
    per_sc = n_ranges // NC
    zone = ((rng_rows + 8 + NS * 8 - 1) // (NS * 8)) * 8
    rng_pad = zone * NS
    zeros = jnp.zeros((zone, dp), f32)
    mesh = plsc.VectorSubcoreMesh(core_axis_name="c", subcore_axis_name="s")

    def body(vec_h, e0_h, z_h, out_h, idx_v, adj_v, vec_v, buf, sem):
        cid = lax.axis_index("c")
        sid = lax.axis_index("s")

        for r in range(per_sc):
            rng = cid * per_sc + r
            base_row = rng * rng_rows
            pltpu.sync_copy(z_h, buf.at[pl.ds(sid * zone, zone)])
            plsc.subcore_barrier()

            def step(j, carry2, base_row=base_row):
                c = j * NS + sid

                @pl.when(c < nchunk)
                def _():
                    base = c * CK
                    cp1 = pltpu.async_copy(e0_h.at[pl.ds(base, CK)], idx_v, sem)
                    cp2 = pltpu.async_copy(vec_h.at[pl.ds(base, CK)], vec_v, sem)
                    cp1.wait()
                    cp2.wait()
                    for k in range(CK // 16):
                        v = lax.shift_right_logical(
                            idx_v[pl.ds(k * 16, 16)], 2) - base_row
                        oob = (v < 0) | (v >= rng_rows)
                        adj_v[pl.ds(k * 16, 16)] = jnp.where(
                            oob, rng_rows + (k % 8), v)
                    pltpu.sync_copy(vec_v, buf.at[adj_v], add=True)

                return carry2

            lax.fori_loop(0, iters, step, None)
            plsc.subcore_barrier()
            pltpu.sync_copy(buf.at[pl.ds(sid * zone, zone)],
                            out_h.at[rng, pl.ds(sid * zone, zone)])
            plsc.subcore_barrier()

    out_type = jax.ShapeDtypeStruct((n_ranges, rng_pad, dp), f32)
    scratch = [pltpu.VMEM((CK,), i32),
               pltpu.VMEM((CK,), i32),
               pltpu.VMEM((CK, dp), f32),
               pltpu.VMEM_SHARED((rng_pad, dp), f32),
               pltpu.SemaphoreType.DMA]
    out = pl.kernel(body, out_type=out_type, mesh=mesh,
                    scratch_types=scratch)(vec_placed, e0, zeros)
    return out[:, :rng_rows, :].reshape(n_edges, d)


def _relu(x):
    return jnp.maximum(x, 0.0)


def _tail(h, refs):
    n = len(refs)
    for i, (w, b) in enumerate(refs):
        h = jnp.dot(h, w[:], preferred_element_type=f32) + b[:]
        if i < n - 1:
            h = _relu(h)
    return h


def _tc_edge_mlps(srcN, dstN, edges, e0E, e1E, epnN, e0, p1, p2, blk=1600):
    n_edges, node_len = srcN.shape
    edge_len = edges.shape[1]
    grid = n_edges // blk
    e0_3d = e0.reshape(grid, 1, blk)

    def body(srcN_r, dstN_r, eE_r, e0_r, e1_r, epn_r, ei_r,
             w11, b11, w12, b12, w13, b13, w14, b14, w15, b15,
             w21, b21, w22, b22, w23, b23, w24, b24, w25, b25,
             outN_r, outE_r):
        h = (jnp.dot(srcN_r[:], w11[pl.ds(0, node_len), :],
                     preferred_element_type=f32)
             + jnp.dot(dstN_r[:], w11[pl.ds(node_len, node_len), :],
                       preferred_element_type=f32)
             + jnp.dot(eE_r[:], w11[pl.ds(2 * node_len, edge_len), :],
                       preferred_element_type=f32)
             + b11[:])
        h = _relu(h)
        outN_r[:] = _tail(h, [(w12, b12), (w13, b13), (w14, b14), (w15, b15)])

        g = (jnp.dot(e0_r[:], w21[pl.ds(0, edge_len), :],
                     preferred_element_type=f32)
             + jnp.dot(e1_r[:], w21[pl.ds(edge_len, edge_len), :],
                       preferred_element_type=f32)
             + jnp.dot(epn_r[:], w21[pl.ds(2 * edge_len, node_len), :],
                       preferred_element_type=f32)
             + b21[:])
        g = _relu(g)
        g = _tail(g, [(w22, b22), (w23, b23), (w24, b24), (w25, b25)])
        m = (ei_r[0, 0, :] % 4)[:, None]
        outE_r[:] = jnp.concatenate(
            [jnp.where(m == k, g, 0.0) for k in range(4)], axis=1)

    def wspec(w):
        return pl.BlockSpec(w.shape, lambda i: (0, 0))

    def rowspec(d):
        return pl.BlockSpec((blk, d), lambda i: (i, 0))

    wb = []
    flat_params = []
    for (w, b) in p1 + p2:
        b2 = b.reshape(1, -1)
        wb += [wspec(w), wspec(b2)]
        flat_params += [w, b2]

    d2 = p2[-1][0].shape[1]
    out_shape = (jax.ShapeDtypeStruct((n_edges, p1[-1][0].shape[1]), f32),
                 jax.ShapeDtypeStruct((n_edges, 4 * d2), f32))
    return pl.pallas_call(
        body,
        grid=(grid,),
        in_specs=[rowspec(node_len), rowspec(node_len), rowspec(edge_len),
                  rowspec(edge_len), rowspec(edge_len), rowspec(node_len),
                  pl.BlockSpec((1, 1, blk), lambda i: (i, 0, 0))] + wb,
        out_specs=(rowspec(p1[-1][0].shape[1]), rowspec(4 * d2)),
        out_shape=out_shape,
    )(srcN, dstN, edges, e0E, e1E, epnN, e0_3d, *flat_params)


def _tc_node_update(nodes, latP, params, blk=1000):
    n_nodes, node_len = nodes.shape
    lat_len = latP.shape[2]
    grid = n_nodes // blk

    def body(nodes_r, lat_r,
             w1, b1, w2, b2, w3, b3, w4, b4, w5, b5, out_r):
        lat = lat_r[0] + lat_r[1]
        h = (jnp.dot(nodes_r[:], w1[pl.ds(0, node_len), :],
                     preferred_element_type=f32)
             + jnp.dot(lat, w1[pl.ds(node_len, lat_len), :],
                       preferred_element_type=f32)
             + b1[:])
        h = _relu(h)
        out_r[:] = _tail(h, [(w2, b2), (w3, b3), (w4, b4), (w5, b5)])

    def wspec(w):
        return pl.BlockSpec(w.shape, lambda i: (0, 0))

    wb = []
    flat_params = []
    for (w, b) in params:
        b2 = b.reshape(1, -1)
        wb += [wspec(w), wspec(b2)]
        flat_params += [w, b2]

    return pl.pallas_call(
        body,
        grid=(grid,),
        in_specs=[pl.BlockSpec((blk, node_len), lambda i: (i, 0)),
                  pl.BlockSpec((NC, blk, lat_len), lambda i: (0, i, 0))] + wb,
        out_specs=pl.BlockSpec((blk, node_len), lambda i: (i, 0)),
        out_shape=jax.ShapeDtypeStruct((n_nodes, params[-1][0].shape[1]), f32),
    )(nodes, latP, *flat_params)


def _tc_edge_update(edges, lat, params, blk=1600):
    n_edges, edge_len = edges.shape
    lat_len = lat.shape[1]
    grid = n_edges // blk

    def body(edges_r, lat_r,
             w1, b1, w2, b2, w3, b3, w4, b4, w5, b5, out_r):
        h = (jnp.dot(edges_r[:], w1[pl.ds(0, edge_len), :],
                     preferred_element_type=f32)
             + jnp.dot(lat_r[:], w1[pl.ds(edge_len, lat_len), :],
                       preferred_element_type=f32)
             + b1[:])
        h = _relu(h)
        out_r[:] = _tail(h, [(w2, b2), (w3, b3), (w4, b4), (w5, b5)])

    def wspec(w):
        return pl.BlockSpec(w.shape, lambda i: (0, 0))

    wb = []
    flat_params = []
    for (w, b) in params:
        b2 = b.reshape(1, -1)
        wb += [wspec(w), wspec(b2)]
        flat_params += [w, b2]

    return pl.pallas_call(
        body,
        grid=(grid,),
        in_specs=[pl.BlockSpec((blk, edge_len), lambda i: (i, 0)),
                  pl.BlockSpec((blk, lat_len), lambda i: (i, 0))] + wb,
        out_specs=pl.BlockSpec((blk, edge_len), lambda i: (i, 0)),
        out_shape=jax.ShapeDtypeStruct((n_edges, params[-1][0].shape[1]), f32),
    )(edges, lat, *flat_params)


def kernel(nodes, edges, edge_index, edge_pair_index, edge_pair_node,
           nodeInt_params, edgeInt_params, nodeUpdate_params,
           edgeUpdate_params):
    n_nodes, node_len = nodes.shape
    edge_len = edges.shape[1]
    src, dst = edge_index[0], edge_index[1]
    e0, e1 = edge_pair_index[0], edge_pair_index[1]

    edges_pad = jnp.pad(edges, ((0, 0), (0, node_len - edge_len)))
    srcN, dstN, epnN, e0E, e1E = _sc_gather(
        nodes, edges_pad, edge_len, src, dst, edge_pair_node, e0, e1)

    nodeIntVec, edgeVecPlaced = _tc_edge_mlps(
        srcN, dstN, edges, e0E, e1E, epnN, e0, nodeInt_params, edgeInt_params)

    nodeLatP = _sc_scatter_node(nodeIntVec, dst, n_nodes)
    edgeLat = _sc_scatter_edge(edgeVecPlaced, e0,
                               edgeInt_params[-1][0].shape[1])

    nodesOut = _tc_node_update(nodes, nodeLatP, nodeUpdate_params)
    edgesOut = _tc_edge_update(edges, edgeLat, edgeUpdate_params)
    return (nodesOut, edgesOut)


# speedup vs baseline: 2.7555x; 1.0677x over previous
import jax
import jax.numpy as jnp
from jax import lax
from jax.experimental import pallas as pl
from jax.experimental.pallas import tpu as pltpu
from jax.experimental.pallas import tpu_sc as plsc
from jax.scipy.linalg import block_diag

NC = 2
NS = 16
NW = NC * NS
CK = 128

f32 = jnp.float32
bf16 = jnp.bfloat16
i32 = jnp.int32


def _sc_gather(nodes, edges_pad, edge_len, src, dst, epn, e0, e1):
    n_nodes, node_len = nodes.shape
    n_edges = edges_pad.shape[0]
    ec = edge_len
    nchunk = n_edges // CK
    iters = pl.cdiv(nchunk, NW)
    mesh = plsc.VectorSubcoreMesh(core_axis_name="c", subcore_axis_name="s")

    def body(nodes_h, edges_h, src_h, dst_h, epn_h, e0_h, e1_h,
             srcN_h, dstN_h, epnN_h, e0E_h, e1E_h,
             isrc, idst, iepn, ie0, ie1,
             rsrc, rdst, repn, re0, re1, ce0, ce1, sem):
        wid = lax.axis_index("s") * NC + lax.axis_index("c")

        def step(j, carry):
            c = j * NW + wid

            @pl.when(c < nchunk)
            def _():
                base = c * CK
                cps = [pltpu.async_copy(src_h.at[pl.ds(base, CK)], isrc, sem),
                       pltpu.async_copy(dst_h.at[pl.ds(base, CK)], idst, sem),
                       pltpu.async_copy(epn_h.at[pl.ds(base, CK)], iepn, sem),
                       pltpu.async_copy(e0_h.at[pl.ds(base, CK)], ie0, sem),
                       pltpu.async_copy(e1_h.at[pl.ds(base, CK)], ie1, sem)]
                for cp in cps:
                    cp.wait()
                cps = [pltpu.async_copy(nodes_h.at[isrc], rsrc, sem),
                       pltpu.async_copy(nodes_h.at[idst], rdst, sem),
                       pltpu.async_copy(nodes_h.at[iepn], repn, sem),
                       pltpu.async_copy(edges_h.at[ie0], re0, sem),
                       pltpu.async_copy(edges_h.at[ie1], re1, sem)]
                for cp in cps:
                    cp.wait()

                def compact(r, carry2):
                    ce0[r, :] = re0[r, pl.ds(0, ec)]
                    ce1[r, :] = re1[r, pl.ds(0, ec)]
                    return carry2

                lax.fori_loop(0, CK, compact, None)
                cps = [pltpu.async_copy(rsrc, srcN_h.at[pl.ds(base, CK)], sem),
                       pltpu.async_copy(rdst, dstN_h.at[pl.ds(base, CK)], sem),
                       pltpu.async_copy(repn, epnN_h.at[pl.ds(base, CK)], sem),
                       pltpu.async_copy(ce0, e0E_h.at[pl.ds(base, CK)], sem),
                       pltpu.async_copy(ce1, e1E_h.at[pl.ds(base, CK)], sem)]
                for cp in cps:
                    cp.wait()

            return carry

        lax.fori_loop(0, iters, step, None)

    out_type = (jax.ShapeDtypeStruct((n_edges, node_len), f32),
                jax.ShapeDtypeStruct((n_edges, node_len), f32),
                jax.ShapeDtypeStruct((n_edges, node_len), f32),
                jax.ShapeDtypeStruct((n_edges, ec), f32),
                jax.ShapeDtypeStruct((n_edges, ec), f32))
    scratch = [pltpu.VMEM((CK,), i32)] * 5 + \
              [pltpu.VMEM((CK, node_len), f32)] * 5 + \
              [pltpu.VMEM((CK, ec), f32)] * 2 + \
              [pltpu.SemaphoreType.DMA]
    return pl.kernel(body, out_type=out_type, mesh=mesh,
                     scratch_types=scratch)(nodes, edges_pad, src, dst,
                                            epn, e0, e1)


def _sc_scatter_node(vec, dst, n_nodes):
    n_edges, d = vec.shape
    nchunk = n_edges // CK
    iters = pl.cdiv(nchunk, NW)
    zone = ((n_nodes + NS * 8 - 1) // (NS * 8)) * 8
    n_pad = zone * NS
    zeros = jnp.zeros((zone, d), f32)
    mesh = plsc.VectorSubcoreMesh(core_axis_name="c", subcore_axis_name="s")

    def body(vec_h, dst_h, z_h, out_h, idx_v, vec_v, buf, sem):
        cid = lax.axis_index("c")
        sid = lax.axis_index("s")
        wid = sid * NC + cid
        pltpu.sync_copy(z_h, buf.at[pl.ds(sid * zone, zone)])
        plsc.subcore_barrier()

        def step(j, carry):
            c = j * NW + wid

            @pl.when(c < nchunk)
            def _():
                base = c * CK
                cp1 = pltpu.async_copy(dst_h.at[pl.ds(base, CK)], idx_v, sem)
                cp2 = pltpu.async_copy(vec_h.at[pl.ds(base, CK)], vec_v, sem)
                cp1.wait()
                cp2.wait()
                pltpu.sync_copy(vec_v, buf.at[idx_v], add=True)

            return carry

        lax.fori_loop(0, iters, step, None)
        plsc.subcore_barrier()
        pltpu.sync_copy(buf.at[pl.ds(sid * zone, zone)],
                        out_h.at[cid, pl.ds(sid * zone, zone)])

    out_type = jax.ShapeDtypeStruct((NC, n_pad, d), f32)
    scratch = [pltpu.VMEM((CK,), i32),
               pltpu.VMEM((CK, d), f32),
               pltpu.VMEM_SHARED((n_pad, d), f32),
               pltpu.SemaphoreType.DMA]
    return pl.kernel(body, out_type=out_type, mesh=mesh,
                     scratch_types=scratch)(vec, dst, zeros)


def _sc_scatter_edge(vec_placed, e0, n_ranges=4):
    n_edges, dp = vec_placed.shape
    nchunk = n_edges // CK
    iters = pl.cdiv(nchunk, NS)
    rng_rows = n_edges // 4 // n_ranges
    per_sc = n_ranges // NC
    zone = ((rng_rows + 8 + NS * 8 - 1) // (NS * 8)) * 8
    rng_pad = zone * NS
    zeros = jnp.zeros((zone, dp), f32)
    mesh = plsc.VectorSubcoreMesh(core_axis_name="c", subcore_axis_name="s")

    def body(vec_h, e0_h, z_h, out_h, idx_v, adj_v, vec_v, buf, sem):
        cid = lax.axis_index("c")
        sid = lax.axis_index("s")

        for r in range(per_sc):
            rng = cid * per_sc + r
            base_row = rng * rng_rows
            pltpu.sync_copy(z_h, buf.at[pl.ds(sid * zone, zone)])
            plsc.subcore_barrier()

            def step(j, carry2, base_row=base_row):
                c = j * NS + sid

                @pl.when(c < nchunk)
                def _():
                    base = c * CK
                    cp1 = pltpu.async_copy(e0_h.at[pl.ds(base, CK)], idx_v, sem)
                    cp2 = pltpu.async_copy(vec_h.at[pl.ds(base, CK)], vec_v, sem)
                    cp1.wait()
                    cp2.wait()
                    for k in range(CK // 16):
                        v = lax.shift_right_logical(
                            idx_v[pl.ds(k * 16, 16)], 2) - base_row
                        oob = (v < 0) | (v >= rng_rows)
                        adj_v[pl.ds(k * 16, 16)] = jnp.where(
                            oob, rng_rows + (k % 8), v)
                    pltpu.sync_copy(vec_v, buf.at[adj_v], add=True)

                return carry2

            lax.fori_loop(0, iters, step, None)
            plsc.subcore_barrier()
            pltpu.sync_copy(buf.at[pl.ds(sid * zone, zone)],
                            out_h.at[rng, pl.ds(sid * zone, zone)])
            plsc.subcore_barrier()

    out_type = jax.ShapeDtypeStruct((n_ranges, rng_pad, dp), f32)
    scratch = [pltpu.VMEM((CK,), i32),
               pltpu.VMEM((CK,), i32),
               pltpu.VMEM((CK, dp), f32),
               pltpu.VMEM_SHARED((rng_pad, dp), f32),
               pltpu.SemaphoreType.DMA]
    return pl.kernel(body, out_type=out_type, mesh=mesh,
                     scratch_types=scratch)(vec_placed, e0, zeros)


def _relu_b(x):
    return jnp.maximum(x, 0.0).astype(bf16)


def _tail(h, refs, out_f32=True):
    n = len(refs)
    for i, (w, b) in enumerate(refs):
        h = jnp.dot(h, w[:], preferred_element_type=f32) + b[:]
        if i < n - 1:
            h = _relu_b(h)
    return h


def _wspec(w):
    return pl.BlockSpec(w.shape, lambda i: (0, 0))


def _flat(params):
    specs, flat = [], []
    for (w, b) in params:
        wb, b2 = w.astype(bf16), b.reshape(1, -1)
        specs += [_wspec(wb), _wspec(b2)]
        flat += [wb, b2]
    return specs, flat


def _tc_edge_mlps(srcN, dstN, edges_b, e0E, e1E, epnN, e0, p1, p2, blk=1600):
    n_edges, node_len = srcN.shape
    ec = e0E.shape[1]
    edge_len = edges_b.shape[1]
    grid = n_edges // blk
    e0_3d = e0.reshape(grid, 1, blk)

    def body(srcN_r, dstN_r, eE_r, e0_r, e1_r, epn_r, ei_r,
             w11, b11, w12, b12, w13, b13, w14, b14, w15, b15,
             w21e0, w21e1, w21n, b21, w22, b22, w23, b23, w24, b24, w25, b25,
             outN_r, outE_r):
        h = (jnp.dot(srcN_r[:].astype(bf16), w11[pl.ds(0, node_len), :],
                     preferred_element_type=f32)
             + jnp.dot(dstN_r[:].astype(bf16), w11[pl.ds(node_len, node_len), :],
                       preferred_element_type=f32)
             + jnp.dot(eE_r[:].astype(bf16), w11[pl.ds(2 * node_len, edge_len), :],
                       preferred_element_type=f32)
             + b11[:])
        h = _relu_b(h)
        outN_r[:] = _tail(h, [(w12, b12), (w13, b13), (w14, b14), (w15, b15)])

        g = (jnp.dot(e0_r[:].astype(bf16), w21e0[:],
                     preferred_element_type=f32)
             + jnp.dot(e1_r[:].astype(bf16), w21e1[:],
                       preferred_element_type=f32)
             + jnp.dot(epn_r[:].astype(bf16), w21n[:],
                       preferred_element_type=f32)
             + b21[:])
        g = _relu_b(g)
        g = _tail(g, [(w22, b22), (w23, b23), (w24, b24), (w25, b25)])
        m = (ei_r[0, 0, :] % 4)[:, None]
        outE_r[:] = jnp.concatenate(
            [jnp.where(m == k, g, 0.0) for k in range(4)], axis=1)

    def rowspec(d):
        return pl.BlockSpec((blk, d), lambda i: (i, 0))

    specs1, flat1 = _flat(p1)
    w21 = p2[0][0].astype(bf16)
    el = edge_len
    w21e0 = w21[:el]
    w21e1 = w21[el:2 * el]
    w21n = w21[2 * el:]
    b21 = p2[0][1].reshape(1, -1)
    specs2, flat2 = _flat(p2[1:])
    wb = specs1 + [_wspec(w21e0), _wspec(w21e1), _wspec(w21n), _wspec(b21)] \
        + specs2
    flat_params = flat1 + [w21e0, w21e1, w21n, b21] + flat2

    d1 = p1[-1][0].shape[1]
    out_shape = (jax.ShapeDtypeStruct((n_edges, d1), f32),
                 jax.ShapeDtypeStruct((n_edges, 128), f32))
    return pl.pallas_call(
        body,
        grid=(grid,),
        in_specs=[rowspec(node_len), rowspec(node_len), rowspec(edge_len),
                  rowspec(ec), rowspec(ec), rowspec(node_len),
                  pl.BlockSpec((1, 1, blk), lambda i: (i, 0, 0))] + wb,
        out_specs=(rowspec(d1), rowspec(128)),
        out_shape=out_shape,
    )(srcN, dstN, edges_b, e0E, e1E, epnN, e0_3d, *flat_params)


def _tc_node_update(nodes, latP, params, blk=1000):
    n_nodes, node_len = nodes.shape
    lat_len = latP.shape[2]
    grid = n_nodes // blk

    def body(nodes_r, lat_r,
             w1, b1, w2, b2, w3, b3, w4, b4, w5, b5, out_r):
        lat = (lat_r[0] + lat_r[1]).astype(bf16)
        h = (jnp.dot(nodes_r[:].astype(bf16), w1[pl.ds(0, node_len), :],
                     preferred_element_type=f32)
             + jnp.dot(lat, w1[pl.ds(node_len, lat_len), :],
                       preferred_element_type=f32)
             + b1[:])
        h = _relu_b(h)
        out_r[:] = _tail(h, [(w2, b2), (w3, b3), (w4, b4), (w5, b5)])

    wb, flat_params = _flat(params)
    return pl.pallas_call(
        body,
        grid=(grid,),
        in_specs=[pl.BlockSpec((blk, node_len), lambda i: (i, 0)),
                  pl.BlockSpec((NC, blk, lat_len), lambda i: (0, i, 0))] + wb,
        out_specs=pl.BlockSpec((blk, node_len), lambda i: (i, 0)),
        out_shape=jax.ShapeDtypeStruct((n_nodes, params[-1][0].shape[1]), f32),
    )(nodes, latP, *flat_params)


def _tc_edge_update_packed(edges4_b, latPad, params, n_ranges=4, blk_p=2000):
    p_rows = edges4_b.shape[0]
    per_rng = p_rows // n_ranges
    grid = p_rows // blk_p
    blocks_per_rng = per_rng // blk_p

    def body(e_r, lat_r, w1e, w1l, b1, w2, b2, w3, b3, w4, b4, w5, b5, out_r):
        h = (jnp.dot(e_r[:], w1e[:], preferred_element_type=f32)
             + jnp.dot(lat_r[0].astype(bf16), w1l[:],
                       preferred_element_type=f32)
             + b1[:])
        h = _relu_b(h)
        out_r[:] = _tail(h, [(w2, b2), (w3, b3), (w4, b4), (w5, b5)])

    el, ll = 16, 32
    w1 = params[0][0]
    w1e = block_diag(*([w1[:el]] * 4)).astype(bf16)
    w1l = block_diag(*([w1[el:el + ll]] * 4)).astype(bf16)
    b1 = jnp.tile(params[0][1], 4).reshape(1, -1)
    wb = [_wspec(w1e), _wspec(w1l), _wspec(b1)]
    flat_params = [w1e, w1l, b1]
    for (w, b) in params[1:]:
        wbd = block_diag(*([w] * 4)).astype(bf16)
        b4x = jnp.tile(b, 4).reshape(1, -1)
        wb += [_wspec(wbd), _wspec(b4x)]
        flat_params += [wbd, b4x]

    d_out = 4 * params[-1][0].shape[1]
    return pl.pallas_call(
        body,
        grid=(grid,),
        in_specs=[pl.BlockSpec((blk_p, edges4_b.shape[1]), lambda i: (i, 0)),
                  pl.BlockSpec((1, blk_p, 128),
                               lambda i: (i // blocks_per_rng,
                                          i % blocks_per_rng, 0))] + wb,
        out_specs=pl.BlockSpec((blk_p, d_out), lambda i: (i, 0)),
        out_shape=jax.ShapeDtypeStruct((p_rows, d_out), f32),
    )(edges4_b, latPad, *flat_params)


def kernel(nodes, edges, edge_index, edge_pair_index, edge_pair_node,
           nodeInt_params, edgeInt_params, nodeUpdate_params,
           edgeUpdate_params):
    n_nodes, node_len = nodes.shape
    n_edges, edge_len = edges.shape
    src, dst = edge_index[0], edge_index[1]
    e0, e1 = edge_pair_index[0], edge_pair_index[1]

    edges_pad = jnp.pad(edges, ((0, 0), (0, node_len - edge_len)))

    srcN, dstN, epnN, e0E, e1E = _sc_gather(
        nodes, edges_pad, edge_len, src, dst, edge_pair_node, e0, e1)

    nodeIntVec, edgeVecPlaced = _tc_edge_mlps(
        srcN, dstN, edges, e0E, e1E, epnN, e0,
        nodeInt_params, edgeInt_params)

    nodeLatP = _sc_scatter_node(nodeIntVec, dst, n_nodes)
    edgeLatPad = _sc_scatter_edge(edgeVecPlaced, e0)

    nodesOut = _tc_node_update(nodes, nodeLatP, nodeUpdate_params)
    edges4_b = edges.astype(bf16).reshape(n_edges // 4, 4 * edge_len)
    edgesOut4 = _tc_edge_update_packed(edges4_b, edgeLatPad,
                                       edgeUpdate_params)
    edgesOut = edgesOut4.reshape(n_edges, edge_len)
    return (nodesOut, edgesOut)


# speedup vs baseline: 2.8008x; 1.0164x over previous
import jax
import jax.numpy as jnp
from jax import lax
from jax.experimental import pallas as pl
from jax.experimental.pallas import tpu as pltpu
from jax.experimental.pallas import tpu_sc as plsc
from jax.scipy.linalg import block_diag

NC = 2
NS = 16
NW = NC * NS
CK = 128

f32 = jnp.float32
bf16 = jnp.bfloat16
i32 = jnp.int32


def _sc_gather(nodes, edges_pad, edge_len, src, dst, epn, e0, e1):
    n_nodes, node_len = nodes.shape
    n_edges = edges_pad.shape[0]
    ec = edge_len
    nchunk = n_edges // CK
    iters = pl.cdiv(nchunk, NW)
    mesh = plsc.VectorSubcoreMesh(core_axis_name="c", subcore_axis_name="s")

    def body(nodes_h, edges_h, src_h, dst_h, epn_h, e0_h, e1_h,
             srcN_h, dstN_h, epnN_h, e0E_h, e1E_h,
             isrc, idst, iepn, ie0, ie1,
             rsrc, rdst, repn, re0, re1, ce0, ce1, sem):
        wid = lax.axis_index("s") * NC + lax.axis_index("c")

        def step(j, carry):
            c = j * NW + wid

            @pl.when(c < nchunk)
            def _():
                base = c * CK
                cps = [pltpu.async_copy(src_h.at[pl.ds(base, CK)], isrc, sem),
                       pltpu.async_copy(dst_h.at[pl.ds(base, CK)], idst, sem),
                       pltpu.async_copy(epn_h.at[pl.ds(base, CK)], iepn, sem),
                       pltpu.async_copy(e0_h.at[pl.ds(base, CK)], ie0, sem),
                       pltpu.async_copy(e1_h.at[pl.ds(base, CK)], ie1, sem)]
                for cp in cps:
                    cp.wait()
                cps = [pltpu.async_copy(nodes_h.at[isrc], rsrc, sem),
                       pltpu.async_copy(nodes_h.at[idst], rdst, sem),
                       pltpu.async_copy(nodes_h.at[iepn], repn, sem),
                       pltpu.async_copy(edges_h.at[ie0], re0, sem),
                       pltpu.async_copy(edges_h.at[ie1], re1, sem)]
                for cp in cps:
                    cp.wait()

                def compact(r, carry2):
                    ce0[r, :] = re0[r, pl.ds(0, ec)]
                    ce1[r, :] = re1[r, pl.ds(0, ec)]
                    return carry2

                lax.fori_loop(0, CK, compact, None)
                cps = [pltpu.async_copy(rsrc, srcN_h.at[pl.ds(base, CK)], sem),
                       pltpu.async_copy(rdst, dstN_h.at[pl.ds(base, CK)], sem),
                       pltpu.async_copy(repn, epnN_h.at[pl.ds(base, CK)], sem),
                       pltpu.async_copy(ce0, e0E_h.at[pl.ds(base, CK)], sem),
                       pltpu.async_copy(ce1, e1E_h.at[pl.ds(base, CK)], sem)]
                for cp in cps:
                    cp.wait()

            return carry

        lax.fori_loop(0, iters, step, None)

    out_type = (jax.ShapeDtypeStruct((n_edges, node_len), f32),
                jax.ShapeDtypeStruct((n_edges, node_len), f32),
                jax.ShapeDtypeStruct((n_edges, node_len), f32),
                jax.ShapeDtypeStruct((n_edges, ec), f32),
                jax.ShapeDtypeStruct((n_edges, ec), f32))
    scratch = [pltpu.VMEM((CK,), i32)] * 5 + \
              [pltpu.VMEM((CK, node_len), f32)] * 5 + \
              [pltpu.VMEM((CK, ec), f32)] * 2 + \
              [pltpu.SemaphoreType.DMA]
    return pl.kernel(body, out_type=out_type, mesh=mesh,
                     scratch_types=scratch)(nodes, edges_pad, src, dst,
                                            epn, e0, e1)


def _sc_scatter_node(vec, dst, n_nodes):
    n_edges, d = vec.shape
    nchunk = n_edges // CK
    iters = pl.cdiv(nchunk, NW)
    zone = ((n_nodes + NS * 8 - 1) // (NS * 8)) * 8
    n_pad = zone * NS
    zeros = jnp.zeros((zone, d), f32)
    mesh = plsc.VectorSubcoreMesh(core_axis_name="c", subcore_axis_name="s")

    def body(vec_h, dst_h, z_h, out_h, idx_v, vec_v, buf, sem):
        cid = lax.axis_index("c")
        sid = lax.axis_index("s")
        wid = sid * NC + cid
        pltpu.sync_copy(z_h, buf.at[pl.ds(sid * zone, zone)])
        plsc.subcore_barrier()

        def step(j, carry):
            c = j * NW + wid

            @pl.when(c < nchunk)
            def _():
                base = c * CK
                cp1 = pltpu.async_copy(dst_h.at[pl.ds(base, CK)], idx_v, sem)
                cp2 = pltpu.async_copy(vec_h.at[pl.ds(base, CK)], vec_v, sem)
                cp1.wait()
                cp2.wait()
                pltpu.sync_copy(vec_v, buf.at[idx_v], add=True)

            return carry

        lax.fori_loop(0, iters, step, None)
        plsc.subcore_barrier()
        pltpu.sync_copy(buf.at[pl.ds(sid * zone, zone)],
                        out_h.at[cid, pl.ds(sid * zone, zone)])

    out_type = jax.ShapeDtypeStruct((NC, n_pad, d), f32)
    scratch = [pltpu.VMEM((CK,), i32),
               pltpu.VMEM((CK, d), f32),
               pltpu.VMEM_SHARED((n_pad, d), f32),
               pltpu.SemaphoreType.DMA]
    return pl.kernel(body, out_type=out_type, mesh=mesh,
                     scratch_types=scratch)(vec, dst, zeros)


def _sc_scatter_edge(vec_placed, e0, n_ranges=4):
    n_edges, dp = vec_placed.shape
    nchunk = n_edges // CK
    iters = pl.cdiv(nchunk, NS)
    rng_rows = n_edges // 4 // n_ranges
    per_sc = n_ranges // NC
    zone = ((rng_rows + 8 + NS * 8 - 1) // (NS * 8)) * 8
    rng_pad = zone * NS
    zeros = jnp.zeros((zone, dp), f32)
    mesh = plsc.VectorSubcoreMesh(core_axis_name="c", subcore_axis_name="s")

    def body(vec_h, e0_h, z_h, out_h, idx_v, adj_v, vec_v, buf, sem):
        cid = lax.axis_index("c")
        sid = lax.axis_index("s")

        for r in range(per_sc):
            rng = cid * per_sc + r
            base_row = rng * rng_rows
            pltpu.sync_copy(z_h, buf.at[pl.ds(sid * zone, zone)])
            plsc.subcore_barrier()

            def step(j, carry2, base_row=base_row):
                c = j * NS + sid

                @pl.when(c < nchunk)
                def _():
                    base = c * CK
                    cp1 = pltpu.async_copy(e0_h.at[pl.ds(base, CK)], idx_v, sem)
                    cp2 = pltpu.async_copy(vec_h.at[pl.ds(base, CK)], vec_v, sem)
                    cp1.wait()
                    cp2.wait()
                    for k in range(CK // 16):
                        v = lax.shift_right_logical(
                            idx_v[pl.ds(k * 16, 16)], 2) - base_row
                        oob = (v < 0) | (v >= rng_rows)
                        adj_v[pl.ds(k * 16, 16)] = jnp.where(
                            oob, rng_rows + (k % 8), v)
                    pltpu.sync_copy(vec_v, buf.at[adj_v], add=True)

                return carry2

            lax.fori_loop(0, iters, step, None)
            plsc.subcore_barrier()
            pltpu.sync_copy(buf.at[pl.ds(sid * zone, zone)],
                            out_h.at[rng, pl.ds(sid * zone, zone)])
            plsc.subcore_barrier()

    out_type = jax.ShapeDtypeStruct((n_ranges, rng_pad, dp), f32)
    scratch = [pltpu.VMEM((CK,), i32),
               pltpu.VMEM((CK,), i32),
               pltpu.VMEM((CK, dp), f32),
               pltpu.VMEM_SHARED((rng_pad, dp), f32),
               pltpu.SemaphoreType.DMA]
    return pl.kernel(body, out_type=out_type, mesh=mesh,
                     scratch_types=scratch)(vec_placed, e0, zeros)


def _relu_b(x):
    return jnp.maximum(x, 0.0).astype(bf16)


def _tail(h, refs, out_f32=True):
    n = len(refs)
    for i, (w, b) in enumerate(refs):
        h = jnp.dot(h, w[:], preferred_element_type=f32) + b[:]
        if i < n - 1:
            h = _relu_b(h)
    return h


def _wspec(w):
    return pl.BlockSpec(w.shape, lambda i: (0, 0))


def _flat(params):
    specs, flat = [], []
    for (w, b) in params:
        wb, b2 = w.astype(bf16), b.reshape(1, -1)
        specs += [_wspec(wb), _wspec(b2)]
        flat += [wb, b2]
    return specs, flat


def _tc_edge_mlps(srcN, dstN, edges_b, e0E, e1E, epnN, e0, p1, p2, blk=3200):
    n_edges, node_len = srcN.shape
    ec = e0E.shape[1]
    edge_len = edges_b.shape[1]
    grid = n_edges // blk
    e0_3d = e0.reshape(grid, 1, blk)

    def body(srcN_r, dstN_r, eE_r, e0_r, e1_r, epn_r, ei_r,
             w11, b11, w12, b12, w13, b13, w14, b14, w15, b15,
             w21e0, w21e1, w21n, b21, w22, b22, w23, b23, w24, b24, w25, b25,
             outN_r, outE_r):
        h = (jnp.dot(srcN_r[:].astype(bf16), w11[pl.ds(0, node_len), :],
                     preferred_element_type=f32)
             + jnp.dot(dstN_r[:].astype(bf16), w11[pl.ds(node_len, node_len), :],
                       preferred_element_type=f32)
             + jnp.dot(eE_r[:].astype(bf16), w11[pl.ds(2 * node_len, edge_len), :],
                       preferred_element_type=f32)
             + b11[:])
        h = _relu_b(h)
        outN_r[:] = _tail(h, [(w12, b12), (w13, b13), (w14, b14), (w15, b15)])

        g = (jnp.dot(e0_r[:].astype(bf16), w21e0[:],
                     preferred_element_type=f32)
             + jnp.dot(e1_r[:].astype(bf16), w21e1[:],
                       preferred_element_type=f32)
             + jnp.dot(epn_r[:].astype(bf16), w21n[:],
                       preferred_element_type=f32)
             + b21[:])
        g = _relu_b(g)
        g = _tail(g, [(w22, b22), (w23, b23), (w24, b24), (w25, b25)])
        m = (ei_r[0, 0, :] % 4)[:, None]
        grp = lax.broadcasted_iota(i32, (blk, 128), 1) // 32
        g4 = jnp.concatenate([g, g, g, g], axis=1)
        outE_r[:] = jnp.where(grp == m, g4, 0.0)

    def rowspec(d):
        return pl.BlockSpec((blk, d), lambda i: (i, 0))

    specs1, flat1 = _flat(p1)
    w21 = p2[0][0].astype(bf16)
    el = edge_len
    w21e0 = w21[:el]
    w21e1 = w21[el:2 * el]
    w21n = w21[2 * el:]
    b21 = p2[0][1].reshape(1, -1)
    specs2, flat2 = _flat(p2[1:])
    wb = specs1 + [_wspec(w21e0), _wspec(w21e1), _wspec(w21n), _wspec(b21)] \
        + specs2
    flat_params = flat1 + [w21e0, w21e1, w21n, b21] + flat2

    d1 = p1[-1][0].shape[1]
    out_shape = (jax.ShapeDtypeStruct((n_edges, d1), f32),
                 jax.ShapeDtypeStruct((n_edges, 128), f32))
    return pl.pallas_call(
        body,
        grid=(grid,),
        in_specs=[rowspec(node_len), rowspec(node_len), rowspec(edge_len),
                  rowspec(ec), rowspec(ec), rowspec(node_len),
                  pl.BlockSpec((1, 1, blk), lambda i: (i, 0, 0))] + wb,
        out_specs=(rowspec(d1), rowspec(128)),
        out_shape=out_shape,
    )(srcN, dstN, edges_b, e0E, e1E, epnN, e0_3d, *flat_params)


def _tc_node_update(nodes, latP, params, blk=1000):
    n_nodes, node_len = nodes.shape
    lat_len = latP.shape[2]
    grid = n_nodes // blk

    def body(nodes_r, lat_r,
             w1, b1, w2, b2, w3, b3, w4, b4, w5, b5, out_r):
        lat = (lat_r[0] + lat_r[1]).astype(bf16)
        h = (jnp.dot(nodes_r[:].astype(bf16), w1[pl.ds(0, node_len), :],
                     preferred_element_type=f32)
             + jnp.dot(lat, w1[pl.ds(node_len, lat_len), :],
                       preferred_element_type=f32)
             + b1[:])
        h = _relu_b(h)
        out_r[:] = _tail(h, [(w2, b2), (w3, b3), (w4, b4), (w5, b5)])

    wb, flat_params = _flat(params)
    return pl.pallas_call(
        body,
        grid=(grid,),
        in_specs=[pl.BlockSpec((blk, node_len), lambda i: (i, 0)),
                  pl.BlockSpec((NC, blk, lat_len), lambda i: (0, i, 0))] + wb,
        out_specs=pl.BlockSpec((blk, node_len), lambda i: (i, 0)),
        out_shape=jax.ShapeDtypeStruct((n_nodes, params[-1][0].shape[1]), f32),
    )(nodes, latP, *flat_params)


def _tc_edge_update_packed(edges4_b, latPad, params, n_ranges=4, blk_p=2000):
    p_rows = edges4_b.shape[0]
    per_rng = p_rows // n_ranges
    grid = p_rows // blk_p
    blocks_per_rng = per_rng // blk_p

    def body(e_r, lat_r, w1e, w1l, b1, w2, b2, w3, b3, w4, b4, w5, b5, out_r):
        h = (jnp.dot(e_r[:], w1e[:], preferred_element_type=f32)
             + jnp.dot(lat_r[0].astype(bf16), w1l[:],
                       preferred_element_type=f32)
             + b1[:])
        h = _relu_b(h)
        out_r[:] = _tail(h, [(w2, b2), (w3, b3), (w4, b4), (w5, b5)])

    el, ll = 16, 32
    w1 = params[0][0]
    w1e = block_diag(*([w1[:el]] * 4)).astype(bf16)
    w1l = block_diag(*([w1[el:el + ll]] * 4)).astype(bf16)
    b1 = jnp.tile(params[0][1], 4).reshape(1, -1)
    wb = [_wspec(w1e), _wspec(w1l), _wspec(b1)]
    flat_params = [w1e, w1l, b1]
    for (w, b) in params[1:]:
        wbd = block_diag(*([w] * 4)).astype(bf16)
        b4x = jnp.tile(b, 4).reshape(1, -1)
        wb += [_wspec(wbd), _wspec(b4x)]
        flat_params += [wbd, b4x]

    d_out = 4 * params[-1][0].shape[1]
    return pl.pallas_call(
        body,
        grid=(grid,),
        in_specs=[pl.BlockSpec((blk_p, edges4_b.shape[1]), lambda i: (i, 0)),
                  pl.BlockSpec((1, blk_p, 128),
                               lambda i: (i // blocks_per_rng,
                                          i % blocks_per_rng, 0))] + wb,
        out_specs=pl.BlockSpec((blk_p, d_out), lambda i: (i, 0)),
        out_shape=jax.ShapeDtypeStruct((p_rows, d_out), f32),
    )(edges4_b, latPad, *flat_params)


def kernel(nodes, edges, edge_index, edge_pair_index, edge_pair_node,
           nodeInt_params, edgeInt_params, nodeUpdate_params,
           edgeUpdate_params):
    n_nodes, node_len = nodes.shape
    n_edges, edge_len = edges.shape
    src, dst = edge_index[0], edge_index[1]
    e0, e1 = edge_pair_index[0], edge_pair_index[1]

    edges_pad = jnp.pad(edges, ((0, 0), (0, node_len - edge_len)))

    srcN, dstN, epnN, e0E, e1E = _sc_gather(
        nodes, edges_pad, edge_len, src, dst, edge_pair_node, e0, e1)

    nodeIntVec, edgeVecPlaced = _tc_edge_mlps(
        srcN, dstN, edges, e0E, e1E, epnN, e0,
        nodeInt_params, edgeInt_params)

    nodeLatP = _sc_scatter_node(nodeIntVec, dst, n_nodes)
    edgeLatPad = _sc_scatter_edge(edgeVecPlaced, e0)

    nodesOut = _tc_node_update(nodes, nodeLatP, nodeUpdate_params)
    edges4_b = edges.astype(bf16).reshape(n_edges // 4, 4 * edge_len)
    edgesOut4 = _tc_edge_update_packed(edges4_b, edgeLatPad,
                                       edgeUpdate_params)
    edgesOut = edgesOut4.reshape(n_edges, edge_len)
    return (nodesOut, edgesOut)


# speedup vs baseline: 3.4043x; 1.2155x over previous
import jax
import jax.numpy as jnp
from jax import lax
from jax.experimental import pallas as pl
from jax.experimental.pallas import tpu as pltpu
from jax.experimental.pallas import tpu_sc as plsc
from jax.scipy.linalg import block_diag

NC = 2
NS = 16
NW = NC * NS
CK = 128

f32 = jnp.float32
bf16 = jnp.bfloat16
i32 = jnp.int32


def _sc_gather(nodes, edges_pad, edge_len, src, dst, epn, e0, e1):
    n_nodes, node_len = nodes.shape
    n_edges = src.shape[0]
    ec = edge_len
    nchunk = n_edges // CK
    iters = pl.cdiv(nchunk, NW)
    mesh = plsc.VectorSubcoreMesh(core_axis_name="c", subcore_axis_name="s")

    def body(nodes_h, edges_h, src_h, dst_h, epn_h, e0_h, e1_h,
             srcN_h, dstN_h, epnN_h, e0E_h, e1E_h,
             isrc, idst, iepn, ie0, ie1,
             rsrc, rdst, repn, re0, re1, ce0, ce1, sem):
        wid = lax.axis_index("s") * NC + lax.axis_index("c")

        def step(j, carry):
            c = j * NW + wid

            @pl.when(c < nchunk)
            def _():
                base = c * CK
                cps = [pltpu.async_copy(src_h.at[pl.ds(base, CK)], isrc, sem),
                       pltpu.async_copy(dst_h.at[pl.ds(base, CK)], idst, sem),
                       pltpu.async_copy(epn_h.at[pl.ds(base, CK)], iepn, sem),
                       pltpu.async_copy(e0_h.at[pl.ds(base, CK)], ie0, sem),
                       pltpu.async_copy(e1_h.at[pl.ds(base, CK)], ie1, sem)]
                for cp in cps:
                    cp.wait()
                cps = [pltpu.async_copy(nodes_h.at[isrc], rsrc, sem),
                       pltpu.async_copy(nodes_h.at[idst], rdst, sem),
                       pltpu.async_copy(nodes_h.at[iepn], repn, sem),
                       pltpu.async_copy(edges_h.at[ie0], re0, sem),
                       pltpu.async_copy(edges_h.at[ie1], re1, sem)]
                for cp in cps:
                    cp.wait()

                def compact(r, carry2):
                    ce0[r, :] = re0[r, pl.ds(0, ec)]
                    ce1[r, :] = re1[r, pl.ds(0, ec)]
                    return carry2

                lax.fori_loop(0, CK, compact, None)
                cps = [pltpu.async_copy(rsrc, srcN_h.at[pl.ds(base, CK)], sem),
                       pltpu.async_copy(rdst, dstN_h.at[pl.ds(base, CK)], sem),
                       pltpu.async_copy(repn, epnN_h.at[pl.ds(base, CK)], sem),
                       pltpu.async_copy(ce0, e0E_h.at[pl.ds(base, CK)], sem),
                       pltpu.async_copy(ce1, e1E_h.at[pl.ds(base, CK)], sem)]
                for cp in cps:
                    cp.wait()

            return carry

        lax.fori_loop(0, iters, step, None)

    out_type = (jax.ShapeDtypeStruct((n_edges, node_len), f32),
                jax.ShapeDtypeStruct((n_edges, node_len), f32),
                jax.ShapeDtypeStruct((n_edges, node_len), f32),
                jax.ShapeDtypeStruct((n_edges, ec), f32),
                jax.ShapeDtypeStruct((n_edges, ec), f32))
    scratch = [pltpu.VMEM((CK,), i32)] * 5 + \
              [pltpu.VMEM((CK, node_len), f32)] * 5 + \
              [pltpu.VMEM((CK, ec), f32)] * 2 + \
              [pltpu.SemaphoreType.DMA]
    return pl.kernel(body, out_type=out_type, mesh=mesh,
                     scratch_types=scratch)(nodes, edges_pad, src, dst,
                                            epn, e0, e1)


def _sc_scatter_node(vec, dst, n_nodes):
    n_edges, d = vec.shape
    nchunk = n_edges // CK
    iters = pl.cdiv(nchunk, NW)
    zone = ((n_nodes + NS * 8 - 1) // (NS * 8)) * 8
    n_pad = zone * NS
    zeros = jnp.zeros((zone, d), f32)
    mesh = plsc.VectorSubcoreMesh(core_axis_name="c", subcore_axis_name="s")

    def body(vec_h, dst_h, z_h, out_h, idx_v, vec_v, buf, sem):
        cid = lax.axis_index("c")
        sid = lax.axis_index("s")
        wid = sid * NC + cid
        pltpu.sync_copy(z_h, buf.at[pl.ds(sid * zone, zone)])
        plsc.subcore_barrier()

        def step(j, carry):
            c = j * NW + wid

            @pl.when(c < nchunk)
            def _():
                base = c * CK
                cp1 = pltpu.async_copy(dst_h.at[pl.ds(base, CK)], idx_v, sem)
                cp2 = pltpu.async_copy(vec_h.at[pl.ds(base, CK)], vec_v, sem)
                cp1.wait()
                cp2.wait()
                pltpu.sync_copy(vec_v, buf.at[idx_v], add=True)

            return carry

        lax.fori_loop(0, iters, step, None)
        plsc.subcore_barrier()
        pltpu.sync_copy(buf.at[pl.ds(sid * zone, zone)],
                        out_h.at[cid, pl.ds(sid * zone, zone)])

    out_type = jax.ShapeDtypeStruct((NC, n_pad, d), f32)
    scratch = [pltpu.VMEM((CK,), i32),
               pltpu.VMEM((CK, d), f32),
               pltpu.VMEM_SHARED((n_pad, d), f32),
               pltpu.SemaphoreType.DMA]
    return pl.kernel(body, out_type=out_type, mesh=mesh,
                     scratch_types=scratch)(vec, dst, zeros)


def _sc_scatter_edge(vec_placed, e0, total_edges, n_ranges=4):
    n_edges, dp = vec_placed.shape
    nchunk = n_edges // CK
    iters = pl.cdiv(nchunk, NS)
    rng_rows = total_edges // 4 // n_ranges
    per_sc = n_ranges // NC
    zone = ((rng_rows + 8 + NS * 8 - 1) // (NS * 8)) * 8
    rng_pad = zone * NS
    zeros = jnp.zeros((zone, dp), f32)
    mesh = plsc.VectorSubcoreMesh(core_axis_name="c", subcore_axis_name="s")

    def body(vec_h, e0_h, z_h, out_h, idx_v, adj_v, vec_v, buf, sem):
        cid = lax.axis_index("c")
        sid = lax.axis_index("s")

        for r in range(per_sc):
            rng = cid * per_sc + r
            base_row = rng * rng_rows
            pltpu.sync_copy(z_h, buf.at[pl.ds(sid * zone, zone)])
            plsc.subcore_barrier()

            def step(j, carry2, base_row=base_row):
                c = j * NS + sid

                @pl.when(c < nchunk)
                def _():
                    base = c * CK
                    cp1 = pltpu.async_copy(e0_h.at[pl.ds(base, CK)], idx_v, sem)
                    cp2 = pltpu.async_copy(vec_h.at[pl.ds(base, CK)], vec_v, sem)
                    cp1.wait()
                    cp2.wait()
                    for k in range(CK // 16):
                        v = lax.shift_right_logical(
                            idx_v[pl.ds(k * 16, 16)], 2) - base_row
                        oob = (v < 0) | (v >= rng_rows)
                        adj_v[pl.ds(k * 16, 16)] = jnp.where(
                            oob, rng_rows + (k % 8), v)
                    pltpu.sync_copy(vec_v, buf.at[adj_v], add=True)

                return carry2

            lax.fori_loop(0, iters, step, None)
            plsc.subcore_barrier()
            pltpu.sync_copy(buf.at[pl.ds(sid * zone, zone)],
                            out_h.at[rng, pl.ds(sid * zone, zone)])
            plsc.subcore_barrier()

    out_type = jax.ShapeDtypeStruct((n_ranges, rng_pad, dp), f32)
    scratch = [pltpu.VMEM((CK,), i32),
               pltpu.VMEM((CK,), i32),
               pltpu.VMEM((CK, dp), f32),
               pltpu.VMEM_SHARED((rng_pad, dp), f32),
               pltpu.SemaphoreType.DMA]
    return pl.kernel(body, out_type=out_type, mesh=mesh,
                     scratch_types=scratch)(vec_placed, e0, zeros)


def _relu_b(x):
    return jnp.maximum(x, 0.0).astype(bf16)


def _tail(h, refs, out_f32=True):
    n = len(refs)
    for i, (w, b) in enumerate(refs):
        h = jnp.dot(h, w[:], preferred_element_type=f32) + b[:]
        if i < n - 1:
            h = _relu_b(h)
    return h


def _wspec(w):
    return pl.BlockSpec(w.shape, lambda i: (0, 0))


def _flat(params):
    specs, flat = [], []
    for (w, b) in params:
        wb, b2 = w.astype(bf16), b.reshape(1, -1)
        specs += [_wspec(wb), _wspec(b2)]
        flat += [wb, b2]
    return specs, flat


def _tc_edge_mlps(srcN, dstN, edges_b, e0E, e1E, epnN, e0, p1, p2, blk=3200):
    n_edges, node_len = srcN.shape
    ec = e0E.shape[1]
    edge_len = edges_b.shape[1]
    grid = n_edges // blk
    e0_3d = e0.reshape(grid, 1, blk)

    def body(srcN_r, dstN_r, eE_r, e0_r, e1_r, epn_r, ei_r,
             w11, b11, w12, b12, w13, b13, w14, b14, w15, b15,
             w21e0, w21e1, w21n, b21, w22, b22, w23, b23, w24, b24, w25, b25,
             outN_r, outE_r):
        h = (jnp.dot(srcN_r[:].astype(bf16), w11[pl.ds(0, node_len), :],
                     preferred_element_type=f32)
             + jnp.dot(dstN_r[:].astype(bf16), w11[pl.ds(node_len, node_len), :],
                       preferred_element_type=f32)
             + jnp.dot(eE_r[:].astype(bf16), w11[pl.ds(2 * node_len, edge_len), :],
                       preferred_element_type=f32)
             + b11[:])
        h = _relu_b(h)
        outN_r[:] = _tail(h, [(w12, b12), (w13, b13), (w14, b14), (w15, b15)])

        g = (jnp.dot(e0_r[:].astype(bf16), w21e0[:],
                     preferred_element_type=f32)
             + jnp.dot(e1_r[:].astype(bf16), w21e1[:],
                       preferred_element_type=f32)
             + jnp.dot(epn_r[:].astype(bf16), w21n[:],
                       preferred_element_type=f32)
             + b21[:])
        g = _relu_b(g)
        g = _tail(g, [(w22, b22), (w23, b23), (w24, b24), (w25, b25)])
        m = (ei_r[0, 0, :] % 4)[:, None]
        grp = lax.broadcasted_iota(i32, (blk, 128), 1) // 32
        g4 = jnp.concatenate([g, g, g, g], axis=1)
        outE_r[:] = jnp.where(grp == m, g4, 0.0)

    def rowspec(d):
        return pl.BlockSpec((blk, d), lambda i: (i, 0))

    specs1, flat1 = _flat(p1)
    w21 = p2[0][0].astype(bf16)
    el = edge_len
    w21e0 = w21[:el]
    w21e1 = w21[el:2 * el]
    w21n = w21[2 * el:]
    b21 = p2[0][1].reshape(1, -1)
    specs2, flat2 = _flat(p2[1:])
    wb = specs1 + [_wspec(w21e0), _wspec(w21e1), _wspec(w21n), _wspec(b21)] \
        + specs2
    flat_params = flat1 + [w21e0, w21e1, w21n, b21] + flat2

    d1 = p1[-1][0].shape[1]
    out_shape = (jax.ShapeDtypeStruct((n_edges, d1), f32),
                 jax.ShapeDtypeStruct((n_edges, 128), f32))
    return pl.pallas_call(
        body,
        grid=(grid,),
        in_specs=[rowspec(node_len), rowspec(node_len), rowspec(edge_len),
                  rowspec(ec), rowspec(ec), rowspec(node_len),
                  pl.BlockSpec((1, 1, blk), lambda i: (i, 0, 0))] + wb,
        out_specs=(rowspec(d1), rowspec(128)),
        out_shape=out_shape,
    )(srcN, dstN, edges_b, e0E, e1E, epnN, e0_3d, *flat_params)


def _tc_node_update(nodes, latPs, params, blk=1000):
    n_nodes, node_len = nodes.shape
    lat_len = latPs[0].shape[2]
    grid = n_nodes // blk
    nl = len(latPs)

    def body(*refs):
        nodes_r = refs[0]
        lat_rs = refs[1:1 + nl]
        (w1, b1, w2, b2, w3, b3, w4, b4, w5, b5) = refs[1 + nl:-1]
        out_r = refs[-1]
        lat = lat_rs[0][0] + lat_rs[0][1]
        for lr in lat_rs[1:]:
            lat = lat + lr[0] + lr[1]
        h = (jnp.dot(nodes_r[:].astype(bf16), w1[pl.ds(0, node_len), :],
                     preferred_element_type=f32)
             + jnp.dot(lat.astype(bf16), w1[pl.ds(node_len, lat_len), :],
                       preferred_element_type=f32)
             + b1[:])
        h = _relu_b(h)
        out_r[:] = _tail(h, [(w2, b2), (w3, b3), (w4, b4), (w5, b5)])

    wb, flat_params = _flat(params)
    latspec = [pl.BlockSpec((NC, blk, lat_len), lambda i: (0, i, 0))] * nl
    return pl.pallas_call(
        body,
        grid=(grid,),
        in_specs=[pl.BlockSpec((blk, node_len), lambda i: (i, 0))] + latspec
        + wb,
        out_specs=pl.BlockSpec((blk, node_len), lambda i: (i, 0)),
        out_shape=jax.ShapeDtypeStruct((n_nodes, params[-1][0].shape[1]), f32),
    )(nodes, *latPs, *flat_params)


def _tc_edge_update_packed(edges4_b, latPads, params, n_ranges=4, blk_p=2000):
    p_rows = edges4_b.shape[0]
    per_rng = p_rows // n_ranges
    grid = p_rows // blk_p
    blocks_per_rng = per_rng // blk_p
    nl = len(latPads)

    def body(*refs):
        e_r = refs[0]
        lat_rs = refs[1:1 + nl]
        (w1e, w1l, b1, w2, b2, w3, b3, w4, b4, w5, b5) = refs[1 + nl:-1]
        out_r = refs[-1]
        lat = lat_rs[0][0]
        for lr in lat_rs[1:]:
            lat = lat + lr[0]
        h = (jnp.dot(e_r[:], w1e[:], preferred_element_type=f32)
             + jnp.dot(lat.astype(bf16), w1l[:],
                       preferred_element_type=f32)
             + b1[:])
        h = _relu_b(h)
        out_r[:] = _tail(h, [(w2, b2), (w3, b3), (w4, b4), (w5, b5)])

    el, ll = 16, 32
    w1 = params[0][0]
    w1e = block_diag(*([w1[:el]] * 4)).astype(bf16)
    w1l = block_diag(*([w1[el:el + ll]] * 4)).astype(bf16)
    b1 = jnp.tile(params[0][1], 4).reshape(1, -1)
    wb = [_wspec(w1e), _wspec(w1l), _wspec(b1)]
    flat_params = [w1e, w1l, b1]
    for (w, b) in params[1:]:
        wbd = block_diag(*([w] * 4)).astype(bf16)
        b4x = jnp.tile(b, 4).reshape(1, -1)
        wb += [_wspec(wbd), _wspec(b4x)]
        flat_params += [wbd, b4x]

    d_out = 4 * params[-1][0].shape[1]
    latspec = [pl.BlockSpec((1, blk_p, 128),
                            lambda i: (i // blocks_per_rng,
                                       i % blocks_per_rng, 0))] * nl
    return pl.pallas_call(
        body,
        grid=(grid,),
        in_specs=[pl.BlockSpec((blk_p, edges4_b.shape[1]), lambda i: (i, 0))]
        + latspec + wb,
        out_specs=pl.BlockSpec((blk_p, d_out), lambda i: (i, 0)),
        out_shape=jax.ShapeDtypeStruct((p_rows, d_out), f32),
    )(edges4_b, *latPads, *flat_params)


def kernel(nodes, edges, edge_index, edge_pair_index, edge_pair_node,
           nodeInt_params, edgeInt_params, nodeUpdate_params,
           edgeUpdate_params):
    n_nodes, node_len = nodes.shape
    n_edges, edge_len = edges.shape
    src, dst = edge_index[0], edge_index[1]
    e0, e1 = edge_pair_index[0], edge_pair_index[1]

    edges_pad = jnp.pad(edges, ((0, 0), (0, node_len - edge_len)))

    H = 2
    eh = n_edges // H
    gathered, mlps = [], []
    for h in range(H):
        sl = slice(h * eh, (h + 1) * eh)
        gathered.append(_sc_gather(nodes, edges_pad, edge_len, src[sl],
                                   dst[sl], edge_pair_node[sl],
                                   e0[sl], e1[sl]))
    for h in range(H):
        sl = slice(h * eh, (h + 1) * eh)
        srcN, dstN, epnN, e0E, e1E = gathered[h]
        mlps.append(_tc_edge_mlps(srcN, dstN, edges[sl], e0E, e1E, epnN,
                                  e0[sl], nodeInt_params, edgeInt_params))

    nodeLatPs, edgeLatPads = [], []
    for h in range(H):
        sl = slice(h * eh, (h + 1) * eh)
        nodeIntVec, edgeVecPlaced = mlps[h]
        nodeLatPs.append(_sc_scatter_node(nodeIntVec, dst[sl], n_nodes))
        edgeLatPads.append(_sc_scatter_edge(edgeVecPlaced, e0[sl], n_edges))

    nodesOut = _tc_node_update(nodes, nodeLatPs, nodeUpdate_params)
    edges4_b = edges.astype(bf16).reshape(n_edges // 4, 4 * edge_len)
    edgesOut4 = _tc_edge_update_packed(edges4_b, edgeLatPads,
                                       edgeUpdate_params)
    edgesOut = edgesOut4.reshape(n_edges, edge_len)
    return (nodesOut, edgesOut)


# speedup vs baseline: 3.4178x; 1.0040x over previous
import jax
import jax.numpy as jnp
from jax import lax
from jax.experimental import pallas as pl
from jax.experimental.pallas import tpu as pltpu
from jax.experimental.pallas import tpu_sc as plsc
from jax.scipy.linalg import block_diag

NC = 2
NS = 16
NW = NC * NS
CK = 128

f32 = jnp.float32
bf16 = jnp.bfloat16
i32 = jnp.int32


def _sc_gather(nodes, edges_pad, edge_len, src, dst, epn, e0, e1,
               row0, n_out):
    n_nodes, node_len = nodes.shape
    n_edges = n_out
    ec = edge_len
    nchunk = n_edges // CK
    iters = pl.cdiv(nchunk, NW)
    mesh = plsc.VectorSubcoreMesh(core_axis_name="c", subcore_axis_name="s")

    def body(nodes_h, edges_h, src_h, dst_h, epn_h, e0_h, e1_h,
             srcN_h, dstN_h, epnN_h, e0E_h, e1E_h,
             isrc, idst, iepn, ie0, ie1,
             rsrc, rdst, repn, re0, re1, ce0, ce1, sem):
        wid = lax.axis_index("s") * NC + lax.axis_index("c")

        def step(j, carry):
            c = j * NW + wid

            @pl.when(c < nchunk)
            def _():
                base = c * CK
                ib = row0 + base
                cps = [pltpu.async_copy(src_h.at[pl.ds(ib, CK)], isrc, sem),
                       pltpu.async_copy(dst_h.at[pl.ds(ib, CK)], idst, sem),
                       pltpu.async_copy(epn_h.at[pl.ds(ib, CK)], iepn, sem),
                       pltpu.async_copy(e0_h.at[pl.ds(ib, CK)], ie0, sem),
                       pltpu.async_copy(e1_h.at[pl.ds(ib, CK)], ie1, sem)]
                for cp in cps:
                    cp.wait()
                cps = [pltpu.async_copy(nodes_h.at[isrc], rsrc, sem),
                       pltpu.async_copy(nodes_h.at[idst], rdst, sem),
                       pltpu.async_copy(nodes_h.at[iepn], repn, sem),
                       pltpu.async_copy(edges_h.at[ie0], re0, sem),
                       pltpu.async_copy(edges_h.at[ie1], re1, sem)]
                for cp in cps:
                    cp.wait()

                def compact(r, carry2):
                    ce0[r, :] = re0[r, pl.ds(0, ec)]
                    ce1[r, :] = re1[r, pl.ds(0, ec)]
                    return carry2

                lax.fori_loop(0, CK, compact, None)
                cps = [pltpu.async_copy(rsrc, srcN_h.at[pl.ds(base, CK)], sem),
                       pltpu.async_copy(rdst, dstN_h.at[pl.ds(base, CK)], sem),
                       pltpu.async_copy(repn, epnN_h.at[pl.ds(base, CK)], sem),
                       pltpu.async_copy(ce0, e0E_h.at[pl.ds(base, CK)], sem),
                       pltpu.async_copy(ce1, e1E_h.at[pl.ds(base, CK)], sem)]
                for cp in cps:
                    cp.wait()

            return carry

        lax.fori_loop(0, iters, step, None)

    out_type = (jax.ShapeDtypeStruct((n_edges, node_len), f32),
                jax.ShapeDtypeStruct((n_edges, node_len), f32),
                jax.ShapeDtypeStruct((n_edges, node_len), f32),
                jax.ShapeDtypeStruct((n_edges, ec), f32),
                jax.ShapeDtypeStruct((n_edges, ec), f32))
    scratch = [pltpu.VMEM((CK,), i32)] * 5 + \
              [pltpu.VMEM((CK, node_len), f32)] * 5 + \
              [pltpu.VMEM((CK, ec), f32)] * 2 + \
              [pltpu.SemaphoreType.DMA]
    return pl.kernel(body, out_type=out_type, mesh=mesh,
                     scratch_types=scratch)(nodes, edges_pad, src, dst,
                                            epn, e0, e1)


def _sc_scatter_node(vec, dst_full, row0, n_nodes, nb=1):
    n_edges, d = vec.shape
    nchunk = n_edges // CK
    iters = pl.cdiv(nchunk, NW * nb)
    zone = ((n_nodes + NS * 8 - 1) // (NS * 8)) * 8
    n_pad = zone * NS
    zeros = jnp.zeros((zone, d), f32)
    mesh = plsc.VectorSubcoreMesh(core_axis_name="c", subcore_axis_name="s")

    def body(vec_h, dst_h, z_h, out_h, *scr):
        idx_vs, vec_vs = scr[:nb], scr[nb:2 * nb]
        buf, sem = scr[2 * nb], scr[2 * nb + 1]
        cid = lax.axis_index("c")
        sid = lax.axis_index("s")
        wid = sid * NC + cid
        pltpu.sync_copy(z_h, buf.at[pl.ds(sid * zone, zone)])
        plsc.subcore_barrier()

        def step(j, carry):
            c = j * NW + wid

            @pl.when(c < nchunk)
            def _():
                base = c * CK
                cp1 = pltpu.async_copy(
                    dst_h.at[pl.ds(row0 + base, CK)], idx_vs[0], sem)
                cp2 = pltpu.async_copy(
                    vec_h.at[pl.ds(base, CK)], vec_vs[0], sem)
                cp1.wait()
                cp2.wait()
                pltpu.sync_copy(vec_vs[0], buf.at[idx_vs[0]], add=True)

            return carry

        lax.fori_loop(0, iters, step, None)
        plsc.subcore_barrier()
        pltpu.sync_copy(buf.at[pl.ds(sid * zone, zone)],
                        out_h.at[cid, pl.ds(sid * zone, zone)])

    out_type = jax.ShapeDtypeStruct((NC, n_pad, d), f32)
    scratch = [pltpu.VMEM((CK,), i32)] * nb + \
              [pltpu.VMEM((CK, d), f32)] * nb + \
              [pltpu.VMEM_SHARED((n_pad, d), f32),
               pltpu.SemaphoreType.DMA]
    return pl.kernel(body, out_type=out_type, mesh=mesh,
                     scratch_types=scratch)(vec, dst_full, zeros)


def _sc_scatter_edge(vec_placed, e0_full, row0, total_edges,
                     n_ranges=4, nb=1):
    n_edges, dp = vec_placed.shape
    nchunk = n_edges // CK
    iters = pl.cdiv(nchunk, NS * nb)
    rng_rows = total_edges // 4 // n_ranges
    per_sc = n_ranges // NC
    zone = ((rng_rows + 8 + NS * 8 - 1) // (NS * 8)) * 8
    rng_pad = zone * NS
    zeros = jnp.zeros((zone, dp), f32)
    mesh = plsc.VectorSubcoreMesh(core_axis_name="c", subcore_axis_name="s")

    def body(vec_h, e0_h, z_h, out_h, *scr):
        idx_vs, adj_vs = scr[:nb], scr[nb:2 * nb]
        vec_vs = scr[2 * nb:3 * nb]
        buf, sem = scr[3 * nb], scr[3 * nb + 1]
        cid = lax.axis_index("c")
        sid = lax.axis_index("s")

        for r in range(per_sc):
            rng = cid * per_sc + r
            base_row = rng * rng_rows
            pltpu.sync_copy(z_h, buf.at[pl.ds(sid * zone, zone)])
            plsc.subcore_barrier()

            def step(j, carry2, base_row=base_row):
                c = j * NS + sid

                @pl.when(c < nchunk)
                def _():
                    base = c * CK
                    cp1 = pltpu.async_copy(
                        e0_h.at[pl.ds(row0 + base, CK)], idx_vs[0], sem)
                    cp2 = pltpu.async_copy(
                        vec_h.at[pl.ds(base, CK)], vec_vs[0], sem)
                    cp1.wait()
                    cp2.wait()
                    for q in range(CK // 16):
                        v = lax.shift_right_logical(
                            idx_vs[0][pl.ds(q * 16, 16)], 2) - base_row
                        oob = (v < 0) | (v >= rng_rows)
                        adj_vs[0][pl.ds(q * 16, 16)] = jnp.where(
                            oob, rng_rows + (q % 8), v)
                    pltpu.sync_copy(vec_vs[0], buf.at[adj_vs[0]], add=True)

                return carry2

            lax.fori_loop(0, iters, step, None)
            plsc.subcore_barrier()
            pltpu.sync_copy(buf.at[pl.ds(sid * zone, zone)],
                            out_h.at[rng, pl.ds(sid * zone, zone)])
            plsc.subcore_barrier()

    out_type = jax.ShapeDtypeStruct((n_ranges, rng_pad, dp), f32)
    scratch = [pltpu.VMEM((CK,), i32)] * (2 * nb) + \
              [pltpu.VMEM((CK, dp), f32)] * nb + \
              [pltpu.VMEM_SHARED((rng_pad, dp), f32),
               pltpu.SemaphoreType.DMA]
    return pl.kernel(body, out_type=out_type, mesh=mesh,
                     scratch_types=scratch)(vec_placed, e0_full, zeros)


def _relu_b(x):
    return jnp.maximum(x, 0.0).astype(bf16)


def _tail(h, refs, out_f32=True):
    n = len(refs)
    for i, (w, b) in enumerate(refs):
        h = jnp.dot(h, w[:], preferred_element_type=f32) + b[:]
        if i < n - 1:
            h = _relu_b(h)
    return h


def _wspec(w):
    return pl.BlockSpec(w.shape, lambda i: (0, 0))


def _flat(params):
    specs, flat = [], []
    for (w, b) in params:
        wb, b2 = w.astype(bf16), b.reshape(1, -1)
        specs += [_wspec(wb), _wspec(b2)]
        flat += [wb, b2]
    return specs, flat


def _tc_edge_mlps(srcN, dstN, edges_b, e0E, e1E, epnN, e0, p1, p2, blk=3200):
    n_edges, node_len = srcN.shape
    ec = e0E.shape[1]
    edge_len = edges_b.shape[1]
    grid = n_edges // blk
    e0_3d = e0.reshape(grid, 1, blk)

    def body(srcN_r, dstN_r, eE_r, e0_r, e1_r, epn_r, ei_r,
             w11, b11, w12, b12, w13, b13, w14, b14, w15, b15,
             w21e0, w21e1, w21n, b21, w22, b22, w23, b23, w24, b24, w25, b25,
             outN_r, outE_r):
        h = (jnp.dot(srcN_r[:].astype(bf16), w11[pl.ds(0, node_len), :],
                     preferred_element_type=f32)
             + jnp.dot(dstN_r[:].astype(bf16), w11[pl.ds(node_len, node_len), :],
                       preferred_element_type=f32)
             + jnp.dot(eE_r[:].astype(bf16), w11[pl.ds(2 * node_len, edge_len), :],
                       preferred_element_type=f32)
             + b11[:])
        h = _relu_b(h)
        outN_r[:] = _tail(h, [(w12, b12), (w13, b13), (w14, b14), (w15, b15)])

        g = (jnp.dot(e0_r[:].astype(bf16), w21e0[:],
                     preferred_element_type=f32)
             + jnp.dot(e1_r[:].astype(bf16), w21e1[:],
                       preferred_element_type=f32)
             + jnp.dot(epn_r[:].astype(bf16), w21n[:],
                       preferred_element_type=f32)
             + b21[:])
        g = _relu_b(g)
        g = _tail(g, [(w22, b22), (w23, b23), (w24, b24), (w25, b25)])
        m = (ei_r[0, 0, :] % 4)[:, None]
        grp = lax.broadcasted_iota(i32, (blk, 128), 1) // 32
        g4 = jnp.concatenate([g, g, g, g], axis=1)
        outE_r[:] = jnp.where(grp == m, g4, 0.0)

    def rowspec(d):
        return pl.BlockSpec((blk, d), lambda i: (i, 0))

    specs1, flat1 = _flat(p1)
    w21 = p2[0][0].astype(bf16)
    el = edge_len
    w21e0 = w21[:el]
    w21e1 = w21[el:2 * el]
    w21n = w21[2 * el:]
    b21 = p2[0][1].reshape(1, -1)
    specs2, flat2 = _flat(p2[1:])
    wb = specs1 + [_wspec(w21e0), _wspec(w21e1), _wspec(w21n), _wspec(b21)] \
        + specs2
    flat_params = flat1 + [w21e0, w21e1, w21n, b21] + flat2

    d1 = p1[-1][0].shape[1]
    out_shape = (jax.ShapeDtypeStruct((n_edges, d1), f32),
                 jax.ShapeDtypeStruct((n_edges, 128), f32))
    return pl.pallas_call(
        body,
        grid=(grid,),
        in_specs=[rowspec(node_len), rowspec(node_len), rowspec(edge_len),
                  rowspec(ec), rowspec(ec), rowspec(node_len),
                  pl.BlockSpec((1, 1, blk), lambda i: (i, 0, 0))] + wb,
        out_specs=(rowspec(d1), rowspec(128)),
        out_shape=out_shape,
    )(srcN, dstN, edges_b, e0E, e1E, epnN, e0_3d, *flat_params)


def _tc_node_update(nodes, latPs, params, blk=1000):
    n_nodes, node_len = nodes.shape
    lat_len = latPs[0].shape[2]
    grid = n_nodes // blk
    nl = len(latPs)

    def body(*refs):
        nodes_r = refs[0]
        lat_rs = refs[1:1 + nl]
        (w1, b1, w2, b2, w3, b3, w4, b4, w5, b5) = refs[1 + nl:-1]
        out_r = refs[-1]
        lat = lat_rs[0][0] + lat_rs[0][1]
        for lr in lat_rs[1:]:
            lat = lat + lr[0] + lr[1]
        h = (jnp.dot(nodes_r[:].astype(bf16), w1[pl.ds(0, node_len), :],
                     preferred_element_type=f32)
             + jnp.dot(lat.astype(bf16), w1[pl.ds(node_len, lat_len), :],
                       preferred_element_type=f32)
             + b1[:])
        h = _relu_b(h)
        out_r[:] = _tail(h, [(w2, b2), (w3, b3), (w4, b4), (w5, b5)])

    wb, flat_params = _flat(params)
    latspec = [pl.BlockSpec((NC, blk, lat_len), lambda i: (0, i, 0))] * nl
    return pl.pallas_call(
        body,
        grid=(grid,),
        in_specs=[pl.BlockSpec((blk, node_len), lambda i: (i, 0))] + latspec
        + wb,
        out_specs=pl.BlockSpec((blk, node_len), lambda i: (i, 0)),
        out_shape=jax.ShapeDtypeStruct((n_nodes, params[-1][0].shape[1]), f32),
    )(nodes, *latPs, *flat_params)


def _tc_edge_update_packed(edges4_b, latPads, params, n_ranges=4, blk_p=2000):
    p_rows = edges4_b.shape[0]
    per_rng = p_rows // n_ranges
    grid = p_rows // blk_p
    blocks_per_rng = per_rng // blk_p
    nl = len(latPads)

    def body(*refs):
        e_r = refs[0]
        lat_rs = refs[1:1 + nl]
        (w1e, w1l, b1, w2, b2, w3, b3, w4, b4, w5, b5) = refs[1 + nl:-1]
        out_r = refs[-1]
        lat = lat_rs[0][0]
        for lr in lat_rs[1:]:
            lat = lat + lr[0]
        h = (jnp.dot(e_r[:], w1e[:], preferred_element_type=f32)
             + jnp.dot(lat.astype(bf16), w1l[:],
                       preferred_element_type=f32)
             + b1[:])
        h = _relu_b(h)
        out_r[:] = _tail(h, [(w2, b2), (w3, b3), (w4, b4), (w5, b5)])

    el, ll = 16, 32
    w1 = params[0][0]
    w1e = block_diag(*([w1[:el]] * 4)).astype(bf16)
    w1l = block_diag(*([w1[el:el + ll]] * 4)).astype(bf16)
    b1 = jnp.tile(params[0][1], 4).reshape(1, -1)
    wb = [_wspec(w1e), _wspec(w1l), _wspec(b1)]
    flat_params = [w1e, w1l, b1]
    for (w, b) in params[1:]:
        wbd = block_diag(*([w] * 4)).astype(bf16)
        b4x = jnp.tile(b, 4).reshape(1, -1)
        wb += [_wspec(wbd), _wspec(b4x)]
        flat_params += [wbd, b4x]

    d_out = 4 * params[-1][0].shape[1]
    latspec = [pl.BlockSpec((1, blk_p, 128),
                            lambda i: (i // blocks_per_rng,
                                       i % blocks_per_rng, 0))] * nl
    return pl.pallas_call(
        body,
        grid=(grid,),
        in_specs=[pl.BlockSpec((blk_p, edges4_b.shape[1]), lambda i: (i, 0))]
        + latspec + wb,
        out_specs=pl.BlockSpec((blk_p, d_out), lambda i: (i, 0)),
        out_shape=jax.ShapeDtypeStruct((p_rows, d_out), f32),
    )(edges4_b, *latPads, *flat_params)


def kernel(nodes, edges, edge_index, edge_pair_index, edge_pair_node,
           nodeInt_params, edgeInt_params, nodeUpdate_params,
           edgeUpdate_params):
    n_nodes, node_len = nodes.shape
    n_edges, edge_len = edges.shape
    src, dst = edge_index[0], edge_index[1]
    e0, e1 = edge_pair_index[0], edge_pair_index[1]

    edges_pad = jnp.pad(edges, ((0, 0), (0, node_len - edge_len)))

    H = 2
    eh = n_edges // H
    gathered, mlps = [], []
    for h in range(H):
        gathered.append(_sc_gather(nodes, edges_pad, edge_len, src,
                                   dst, edge_pair_node, e0, e1,
                                   row0=h * eh, n_out=eh))
    for h in range(H):
        sl = slice(h * eh, (h + 1) * eh)
        srcN, dstN, epnN, e0E, e1E = gathered[h]
        mlps.append(_tc_edge_mlps(srcN, dstN, edges[sl], e0E, e1E, epnN,
                                  e0[sl], nodeInt_params, edgeInt_params))

    nodeLatPs, edgeLatPads = [], []
    for h in range(H):
        nodeIntVec, edgeVecPlaced = mlps[h]
        nodeLatPs.append(_sc_scatter_node(nodeIntVec, dst, h * eh, n_nodes))
        edgeLatPads.append(_sc_scatter_edge(edgeVecPlaced, e0, h * eh,
                                            n_edges))

    nodesOut = _tc_node_update(nodes, nodeLatPs, nodeUpdate_params)
    edges4_b = edges.astype(bf16).reshape(n_edges // 4, 4 * edge_len)
    edgesOut4 = _tc_edge_update_packed(edges4_b, edgeLatPads,
                                       edgeUpdate_params)
    edgesOut = edgesOut4.reshape(n_edges, edge_len)
    return (nodesOut, edgesOut)
